# Initial kernel scaffold; baseline (speedup 1.0000x reference)
import jax
import jax.numpy as jnp
from jax.experimental import pallas as pl


def kernel(indices, table):
    raise NotImplementedError("write your pallas kernel here")


# speedup vs baseline: 1.7984x; 1.7984x over previous
import functools

import jax
import jax.numpy as jnp
from jax import lax
from jax.experimental import pallas as pl
from jax.experimental.pallas import tpu as pltpu
from jax.experimental.pallas import tpu_sc as plsc

NC = 2
NS = 16
NW = NC * NS

CHUNK = 512


def _gather_call(flat_idx, table, b_per_w, n_chunks, dim):
    mesh = plsc.VectorSubcoreMesh(core_axis_name="c", subcore_axis_name="s")

    @functools.partial(
        pl.kernel,
        mesh=mesh,
        out_type=jax.ShapeDtypeStruct((flat_idx.shape[0], dim), jnp.float32),
        scratch_types=[
            pltpu.VMEM((CHUNK,), jnp.int32),
            pltpu.VMEM((CHUNK, dim), jnp.float32),
            pltpu.SemaphoreType.DMA,
        ],
        compiler_params=pltpu.CompilerParams(use_tc_tiling_on_sc=False),
    )
    def run(idx_hbm, table_hbm, out_hbm, idx_v, rows_v, sem):
        wid = lax.axis_index("s") * NC + lax.axis_index("c")
        base = wid * b_per_w

        def step(g, carry):
            off = pl.multiple_of(base + g * CHUNK, CHUNK)
            pltpu.sync_copy(idx_hbm.at[pl.ds(off, CHUNK)], idx_v)
            pltpu.async_copy(table_hbm.at[idx_v], rows_v, sem).wait()
            pltpu.sync_copy(rows_v, out_hbm.at[pl.ds(off, CHUNK)])
            return carry

        lax.fori_loop(0, n_chunks, step, 0)

    return run(flat_idx, table)


def kernel(indices, table):
    out_shape = indices.shape + (table.shape[1],)
    flat_idx = indices.reshape(-1).astype(jnp.int32)
    b = flat_idx.shape[0]
    b_per_w = b // NW
    n_chunks = b_per_w // CHUNK
    out = _gather_call(flat_idx, table, b_per_w, n_chunks, table.shape[1])
    return out.reshape(out_shape)


# speedup vs baseline: 1.8691x; 1.0393x over previous
import functools

import jax
import jax.numpy as jnp
from jax import lax
from jax.experimental import pallas as pl
from jax.experimental.pallas import tpu as pltpu
from jax.experimental.pallas import tpu_sc as plsc

NC = 2
NS = 16
NW = NC * NS

CHUNK = 512


def _gather_call(flat_idx, table, b_per_w, n_chunks, dim):
    mesh = plsc.VectorSubcoreMesh(core_axis_name="c", subcore_axis_name="s")
    n_pairs = n_chunks // 2

    @functools.partial(
        pl.kernel,
        mesh=mesh,
        out_type=jax.ShapeDtypeStruct((flat_idx.shape[0], dim), jnp.float32),
        scratch_types=[
            pltpu.VMEM((b_per_w,), jnp.int32),
            pltpu.VMEM((2, CHUNK, dim), jnp.float32),
            pltpu.SemaphoreType.DMA((2,)),
            pltpu.SemaphoreType.DMA((2,)),
        ],
        compiler_params=pltpu.CompilerParams(use_tc_tiling_on_sc=False),
    )
    def run(idx_hbm, table_hbm, out_hbm, idx_v, rows_v, gsem, osem):
        wid = lax.axis_index("s") * NC + lax.axis_index("c")
        base = wid * b_per_w
        pltpu.sync_copy(idx_hbm.at[pl.ds(base, b_per_w)], idx_v)

        def start_gather(g, slot):
            pltpu.async_copy(
                table_hbm.at[idx_v.at[pl.ds(g * CHUNK, CHUNK)]],
                rows_v.at[slot],
                gsem.at[slot],
            )

        def start_write(g, slot):
            off = pl.multiple_of(base + g * CHUNK, CHUNK)
            pltpu.async_copy(rows_v.at[slot], out_hbm.at[pl.ds(off, CHUNK)],
                             osem.at[slot])

        def wait_gather(slot):
            pltpu.make_async_copy(table_hbm.at[pl.ds(0, CHUNK)],
                                  rows_v.at[slot], gsem.at[slot]).wait()

        def wait_write(slot):
            pltpu.make_async_copy(rows_v.at[slot],
                                  out_hbm.at[pl.ds(0, CHUNK)],
                                  osem.at[slot]).wait()

        start_gather(0, 0)

        def pair(p, carry):
            g0 = p * 2
            wait_gather(0)

            @pl.when(p > 0)
            def _():
                wait_write(1)

            start_gather(g0 + 1, 1)
            start_write(g0, 0)
            wait_gather(1)

            @pl.when(p + 1 < n_pairs)
            def _():
                wait_write(0)
                start_gather(g0 + 2, 0)

            start_write(g0 + 1, 1)
            return carry

        lax.fori_loop(0, n_pairs, pair, 0)
        wait_write(0)
        wait_write(1)

    return run(flat_idx, table)


def kernel(indices, table):
    out_shape = indices.shape + (table.shape[1],)
    flat_idx = indices.reshape(-1).astype(jnp.int32)
    b = flat_idx.shape[0]
    b_per_w = b // NW
    n_chunks = b_per_w // CHUNK
    out = _gather_call(flat_idx, table, b_per_w, n_chunks, table.shape[1])
    return out.reshape(out_shape)


# speedup vs baseline: 1.9548x; 1.0458x over previous
import functools

import jax
import jax.numpy as jnp
from jax import lax
from jax.experimental import pallas as pl
from jax.experimental.pallas import tpu as pltpu
from jax.experimental.pallas import tpu_sc as plsc

NC = 2
NS = 16
NW = NC * NS


def _gather_call(idx_t, table, dim):
    s_len, b_len = idx_t.shape
    b_per_w = b_len // NW
    n_pairs = s_len // 2
    mesh = plsc.VectorSubcoreMesh(core_axis_name="c", subcore_axis_name="s")

    @functools.partial(
        pl.kernel,
        mesh=mesh,
        out_type=jax.ShapeDtypeStruct((s_len, b_len, dim), jnp.float32),
        scratch_types=[
            pltpu.VMEM((s_len, b_per_w), jnp.int32),
            pltpu.VMEM((2, b_per_w, dim), jnp.float32),
            pltpu.SemaphoreType.DMA((2,)),
            pltpu.SemaphoreType.DMA((2,)),
        ],
        compiler_params=pltpu.CompilerParams(use_tc_tiling_on_sc=False),
    )
    def run(idx_hbm, table_hbm, out_hbm, idx_v, rows_v, gsem, osem):
        wid = lax.axis_index("s") * NC + lax.axis_index("c")
        base = pl.multiple_of(wid * b_per_w, b_per_w)
        pltpu.sync_copy(idx_hbm.at[:, pl.ds(base, b_per_w)], idx_v)

        def start_gather(s, slot):
            pltpu.async_copy(
                table_hbm.at[idx_v.at[s]],
                rows_v.at[slot],
                gsem.at[slot],
            )

        def start_write(s, slot):
            pltpu.async_copy(rows_v.at[slot],
                             out_hbm.at[s, pl.ds(base, b_per_w)],
                             osem.at[slot])

        def wait_gather(slot):
            pltpu.make_async_copy(table_hbm.at[pl.ds(0, b_per_w)],
                                  rows_v.at[slot], gsem.at[slot]).wait()

        def wait_write(slot):
            pltpu.make_async_copy(rows_v.at[slot],
                                  out_hbm.at[0, pl.ds(0, b_per_w)],
                                  osem.at[slot]).wait()

        start_gather(0, 0)

        def pair(p, carry):
            s0 = p * 2
            wait_gather(0)

            @pl.when(p > 0)
            def _():
                wait_write(1)

            start_gather(s0 + 1, 1)
            start_write(s0, 0)
            wait_gather(1)

            @pl.when(p + 1 < n_pairs)
            def _():
                wait_write(0)
                start_gather(s0 + 2, 0)

            start_write(s0 + 1, 1)
            return carry

        lax.fori_loop(0, n_pairs, pair, 0)
        wait_write(0)
        wait_write(1)

    return run(idx_t, table)


def kernel(indices, table):
    idx_t = indices.T.astype(jnp.int32)
    out_l = _gather_call(idx_t, table, table.shape[1])
    return jnp.transpose(out_l, (1, 0, 2))


# speedup vs baseline: 2.4155x; 1.2357x over previous
import functools

import jax
import jax.numpy as jnp
from jax import lax
from jax.experimental import pallas as pl
from jax.experimental.pallas import tpu as pltpu
from jax.experimental.pallas import tpu_sc as plsc

NC = 2
NS = 16
NW = NC * NS

CHUNK = 400
PD = 128


def _gather_call(flat_idx, table_p):
    b = flat_idx.shape[0]
    b_per_w = b // NW
    n_pairs = b_per_w // CHUNK // 2
    mesh = plsc.VectorSubcoreMesh(core_axis_name="c", subcore_axis_name="s")

    @functools.partial(
        pl.kernel,
        mesh=mesh,
        out_type=jax.ShapeDtypeStruct((b, PD), jnp.float32),
        scratch_types=[
            pltpu.VMEM((b_per_w,), jnp.int32),
            pltpu.VMEM((2, CHUNK, PD), jnp.float32),
            pltpu.SemaphoreType.DMA((2,)),
            pltpu.SemaphoreType.DMA((2,)),
        ],
        compiler_params=pltpu.CompilerParams(use_tc_tiling_on_sc=True),
    )
    def run(idx_hbm, table_hbm, out_hbm, idx_v, rows_v, gsem, osem):
        wid = lax.axis_index("s") * NC + lax.axis_index("c")
        base = pl.multiple_of(wid * b_per_w, b_per_w)
        pltpu.sync_copy(idx_hbm.at[pl.ds(base, b_per_w)], idx_v)

        def start_gather(g, slot):
            pltpu.async_copy(
                table_hbm.at[idx_v.at[pl.ds(g * CHUNK, CHUNK)]],
                rows_v.at[slot],
                gsem.at[slot],
            )

        def start_write(g, slot):
            off = pl.multiple_of(base + g * CHUNK, CHUNK)
            pltpu.async_copy(rows_v.at[slot], out_hbm.at[pl.ds(off, CHUNK)],
                             osem.at[slot])

        def wait_gather(slot):
            pltpu.make_async_copy(table_hbm.at[pl.ds(0, CHUNK)],
                                  rows_v.at[slot], gsem.at[slot]).wait()

        def wait_write(slot):
            pltpu.make_async_copy(rows_v.at[slot],
                                  out_hbm.at[pl.ds(0, CHUNK)],
                                  osem.at[slot]).wait()

        start_gather(0, 0)

        def pair(p, carry):
            g0 = p * 2
            wait_gather(0)

            @pl.when(p > 0)
            def _():
                wait_write(1)

            start_gather(g0 + 1, 1)
            start_write(g0, 0)
            wait_gather(1)

            @pl.when(p + 1 < n_pairs)
            def _():
                wait_write(0)
                start_gather(g0 + 2, 0)

            start_write(g0 + 1, 1)
            return carry

        lax.fori_loop(0, n_pairs, pair, 0)
        wait_write(0)
        wait_write(1)

    return run(flat_idx, table_p)


def kernel(indices, table):
    nb, ns = indices.shape
    dim = table.shape[1]
    flat_idx = indices.T.reshape(-1).astype(jnp.int32)
    table_p = jnp.pad(table, ((0, 0), (0, PD - dim)))
    out_p = _gather_call(flat_idx, table_p)
    return out_p.reshape(ns, nb, PD)[:, :, :dim].transpose(1, 0, 2)
